# baseline (speedup 1.0000x reference)
import jax
import jax.numpy as jnp
from jax.experimental import pallas as pl

_NSAMPLE = 32
_TEMP = 0.02
_BM = 128
_NC = 512


def _qg_kernel(rt_ref, xyzT_ref, fx_ref, q_ref, r_ref, rr_ref, nf_ref, w_ref,
               idx_ref):
    nch = xyzT_ref.shape[1]
    bm = q_ref.shape[1]
    rt = rt_ref[0, 0]
    q3 = q_ref[0]
    qx = q_ref[0, :, 0:1]
    qy = q_ref[0, :, 1:2]
    qz = q_ref[0, :, 2:3]
    r = r_ref[0, :, 0:1]
    rr = rr_ref[0, :, 0:1]
    rr2 = rr * rr
    qq = qx * qx + qy * qy + qz * qz

    nf_ref[...] = jnp.zeros_like(nf_ref)
    idx_ref[...] = jnp.zeros_like(idx_ref)

    jloc = jax.lax.broadcasted_iota(jnp.int32, (1, _NC), 1)

    def chunk_body(state):
        c, carry = state
        base = c * _NC
        p3 = xyzT_ref[0, c]
        px = p3[0, :][None, :]
        py = p3[1, :][None, :]
        pz = p3[2, :][None, :]
        fxc = fx_ref[0, c]
        pp = px * px + py * py + pz * pz
        dot = jax.lax.dot_general(
            q3, p3, (((1,), (0,)), ((), ())),
            preferred_element_type=jnp.float32)
        d2 = qq + pp - 2.0 * dot
        mask = d2 < rr2
        mi = mask.astype(jnp.int32)
        incl = mi
        sh = 1
        while sh < _NC:
            zeros = jnp.zeros((bm, sh), jnp.int32)
            incl = incl + jnp.concatenate([zeros, incl[:, :_NC - sh]], axis=1)
            sh *= 2
        excl = incl - mi + carry
        jglob = jloc + base
        for s in range(_NSAMPLE):
            hit = mask & (excl == s)
            ind = jnp.where(hit, 1.0, 0.0)
            rawx = jax.lax.dot_general(
                ind, p3, (((1,), (1,)), ((), ())),
                precision=jax.lax.Precision.HIGHEST,
                preferred_element_type=jnp.float32)
            rawf = jax.lax.dot_general(
                ind, fxc, (((1,), (0,)), ((), ())),
                preferred_element_type=jnp.float32)
            nf_ref[:, s, :] += jnp.concatenate([rawx, rawf], axis=1)
            idx_ref[:, s] += jnp.sum(
                jnp.where(hit, jglob, 0), axis=1)
        carry = carry + incl[:, _NC - 1:_NC]
        return c + 1, carry

    def chunk_cond(state):
        c, carry = state
        return (c < nch) & (jnp.min(carry) < _NSAMPLE)

    carry0 = jnp.zeros((bm, 1), jnp.int32)
    _, cnt = jax.lax.while_loop(chunk_cond, chunk_body,
                                (jnp.int32(0), carry0))

    empty = cnt == 0
    raw0 = nf_ref[:, 0, :]
    id0 = idx_ref[:, 0]
    for s in range(_NSAMPLE):
        valid = s < jnp.maximum(cnt, 1)
        sel = jnp.where(valid, nf_ref[:, s, :], raw0)
        gx = jnp.where(empty, 0.0, sel[:, 0:1] - qx)
        gy = jnp.where(empty, 0.0, sel[:, 1:2] - qy)
        gz = jnp.where(empty, 0.0, sel[:, 2:3] - qz)
        dist = jnp.sqrt(gx * gx + gy * gy + gz * gz)
        w = 1.0 - jax.nn.sigmoid((dist - r) / rt)
        out = jnp.concatenate([gx, gy, gz, sel[:, 3:]], axis=1)
        nf_ref[:, s, :] = out
        w_ref[:, s] = w[:, 0]
        idx_ref[:, s] = jnp.where(valid[:, 0], idx_ref[:, s], id0)


def kernel(xyz, xyz_batch_cnt, new_xyz, new_xyz_r, new_xyz_batch_cnt,
           features, temperature_decay):
    b = xyz_batch_cnt.shape[0]
    n = xyz.shape[0] // b
    m = new_xyz.shape[0] // b
    ns = _NSAMPLE
    cdim = 3 + features.shape[1]
    real_t = _TEMP * temperature_decay
    explore_r = new_xyz_r + real_t * 5.0
    rt = jnp.reshape(jnp.asarray(real_t, jnp.float32), (1, 1))
    nch = n // _NC
    xyzT = (xyz.reshape(b, n, 3).transpose(0, 2, 1)
            .reshape(b, 3, nch, _NC).transpose(0, 2, 1, 3))
    fx = features.reshape(b, nch, _NC, cdim - 3)
    qb = new_xyz.reshape(b, m, 3)
    rb = new_xyz_r.reshape(b, m, 1)
    rrb = explore_r.reshape(b, m, 1)
    nblk = m // _BM
    nf_t, w, idx = pl.pallas_call(
        _qg_kernel,
        grid=(b, nblk),
        in_specs=[
            pl.BlockSpec((1, 1), lambda bi, mi: (0, 0)),
            pl.BlockSpec((1, nch, 3, _NC), lambda bi, mi: (bi, 0, 0, 0)),
            pl.BlockSpec((1, nch, _NC, cdim - 3), lambda bi, mi: (bi, 0, 0, 0)),
            pl.BlockSpec((1, _BM, 3), lambda bi, mi: (bi, mi, 0)),
            pl.BlockSpec((1, _BM, 1), lambda bi, mi: (bi, mi, 0)),
            pl.BlockSpec((1, _BM, 1), lambda bi, mi: (bi, mi, 0)),
        ],
        out_specs=[
            pl.BlockSpec((_BM, ns, cdim), lambda bi, mi: (bi * nblk + mi, 0, 0)),
            pl.BlockSpec((_BM, ns), lambda bi, mi: (bi * nblk + mi, 0)),
            pl.BlockSpec((_BM, ns), lambda bi, mi: (bi * nblk + mi, 0)),
        ],
        out_shape=[
            jax.ShapeDtypeStruct((b * m, ns, cdim), jnp.float32),
            jax.ShapeDtypeStruct((b * m, ns), jnp.float32),
            jax.ShapeDtypeStruct((b * m, ns), jnp.int32),
        ],
    )(rt, xyzT, fx, qb, rb, rrb)
    new_features = nf_t.transpose(0, 2, 1)
    return new_features, w, idx


# speedup vs baseline: 1.1133x; 1.1133x over previous
import jax
import jax.numpy as jnp
from jax.experimental import pallas as pl

_NSAMPLE = 32
_TEMP = 0.02
_BM = 128
_NC = 1024


def _qg_kernel(rt_ref, xyzT_ref, fx_ref, q_ref, r_ref, rr_ref, nf_ref, w_ref,
               idx_ref):
    nch = xyzT_ref.shape[1]
    bm = q_ref.shape[1]
    rt = rt_ref[0, 0]
    q3 = q_ref[0]
    qx = q_ref[0, :, 0:1]
    qy = q_ref[0, :, 1:2]
    qz = q_ref[0, :, 2:3]
    r = r_ref[0, :, 0:1]
    rr = rr_ref[0, :, 0:1]
    rr2 = rr * rr
    qq = qx * qx + qy * qy + qz * qz

    nf_ref[...] = jnp.zeros_like(nf_ref)
    idx_ref[...] = jnp.zeros_like(idx_ref)

    jloc = jax.lax.broadcasted_iota(jnp.int32, (1, _NC), 1)

    def chunk_body(state):
        c, carry = state
        base = c * _NC
        p3 = xyzT_ref[0, c]
        px = p3[0, :][None, :]
        py = p3[1, :][None, :]
        pz = p3[2, :][None, :]
        fxc = fx_ref[0, c]
        pp = px * px + py * py + pz * pz
        dot = jax.lax.dot_general(
            q3, p3, (((1,), (0,)), ((), ())),
            preferred_element_type=jnp.float32)
        d2 = qq + pp - 2.0 * dot
        mask = d2 < rr2
        mi = mask.astype(jnp.int32)
        incl = mi
        sh = 1
        while sh < _NC:
            zeros = jnp.zeros((bm, sh), jnp.int32)
            incl = incl + jnp.concatenate([zeros, incl[:, :_NC - sh]], axis=1)
            sh *= 2
        excl = incl - mi + carry
        jglob = jloc + base
        for s in range(_NSAMPLE):
            hit = mask & (excl == s)
            ind = jnp.where(hit, 1.0, 0.0)
            raw = jax.lax.dot_general(
                ind, fxc, (((1,), (0,)), ((), ())),
                precision=jax.lax.Precision.HIGHEST,
                preferred_element_type=jnp.float32)
            nf_ref[:, s, :] += raw
            idx_ref[:, s] += jnp.sum(
                jnp.where(hit, jglob, 0), axis=1)
        carry = carry + incl[:, _NC - 1:_NC]
        return c + 1, carry

    def chunk_cond(state):
        c, carry = state
        return (c < nch) & (jnp.min(carry) < _NSAMPLE)

    carry0 = jnp.zeros((bm, 1), jnp.int32)
    _, cnt = jax.lax.while_loop(chunk_cond, chunk_body,
                                (jnp.int32(0), carry0))

    empty = cnt == 0
    raw0 = nf_ref[:, 0, :]
    id0 = idx_ref[:, 0]
    for s in range(_NSAMPLE):
        valid = s < jnp.maximum(cnt, 1)
        sel = jnp.where(valid, nf_ref[:, s, :], raw0)
        gx = jnp.where(empty, 0.0, sel[:, 0:1] - qx)
        gy = jnp.where(empty, 0.0, sel[:, 1:2] - qy)
        gz = jnp.where(empty, 0.0, sel[:, 2:3] - qz)
        dist = jnp.sqrt(gx * gx + gy * gy + gz * gz)
        w = 1.0 - jax.nn.sigmoid((dist - r) / rt)
        out = jnp.concatenate([gx, gy, gz, sel[:, 3:]], axis=1)
        nf_ref[:, s, :] = out
        w_ref[:, s] = w[:, 0]
        idx_ref[:, s] = jnp.where(valid[:, 0], idx_ref[:, s], id0)


def kernel(xyz, xyz_batch_cnt, new_xyz, new_xyz_r, new_xyz_batch_cnt,
           features, temperature_decay):
    b = xyz_batch_cnt.shape[0]
    n = xyz.shape[0] // b
    m = new_xyz.shape[0] // b
    ns = _NSAMPLE
    cdim = 3 + features.shape[1]
    real_t = _TEMP * temperature_decay
    explore_r = new_xyz_r + real_t * 5.0
    rt = jnp.reshape(jnp.asarray(real_t, jnp.float32), (1, 1))
    nch = n // _NC
    xyzT = (xyz.reshape(b, n, 3).transpose(0, 2, 1)
            .reshape(b, 3, nch, _NC).transpose(0, 2, 1, 3))
    fx = (jnp.concatenate([xyz, features], axis=1)
          .reshape(b, nch, _NC, cdim))
    qb = new_xyz.reshape(b, m, 3)
    rb = new_xyz_r.reshape(b, m, 1)
    rrb = explore_r.reshape(b, m, 1)
    nblk = m // _BM
    nf_t, w, idx = pl.pallas_call(
        _qg_kernel,
        grid=(b, nblk),
        in_specs=[
            pl.BlockSpec((1, 1), lambda bi, mi: (0, 0)),
            pl.BlockSpec((1, nch, 3, _NC), lambda bi, mi: (bi, 0, 0, 0)),
            pl.BlockSpec((1, nch, _NC, cdim), lambda bi, mi: (bi, 0, 0, 0)),
            pl.BlockSpec((1, _BM, 3), lambda bi, mi: (bi, mi, 0)),
            pl.BlockSpec((1, _BM, 1), lambda bi, mi: (bi, mi, 0)),
            pl.BlockSpec((1, _BM, 1), lambda bi, mi: (bi, mi, 0)),
        ],
        out_specs=[
            pl.BlockSpec((_BM, ns, cdim), lambda bi, mi: (bi * nblk + mi, 0, 0)),
            pl.BlockSpec((_BM, ns), lambda bi, mi: (bi * nblk + mi, 0)),
            pl.BlockSpec((_BM, ns), lambda bi, mi: (bi * nblk + mi, 0)),
        ],
        out_shape=[
            jax.ShapeDtypeStruct((b * m, ns, cdim), jnp.float32),
            jax.ShapeDtypeStruct((b * m, ns), jnp.float32),
            jax.ShapeDtypeStruct((b * m, ns), jnp.int32),
        ],
    )(rt, xyzT, fx, qb, rb, rrb)
    new_features = nf_t.transpose(0, 2, 1)
    return new_features, w, idx
